# baseline (speedup 1.0000x reference)
import functools

import jax
import jax.numpy as jnp
import numpy as np
from jax import lax
from jax.experimental import pallas as pl
from jax.experimental.pallas import tpu as pltpu
from jax.experimental.pallas import tpu_sc as plsc

B = 16384
D = 16
P = 25
EPS = 1e-5
NQ = 12
NS_ = 13
RT = 1280
SLOTS = 32

_info = plsc.get_sparse_core_info()
NC, NCS, L = _info.num_cores, _info.num_subcores, _info.num_lanes
NW = NC * NCS
BPW = B // NW
G = BPW // L


def _pair_const():
    m = np.zeros((50, SLOTS), np.float32)
    offs = np.zeros((SLOTS,), np.float32)
    for br in range(2):
        for q in range(NQ):
            m[br * P + 2 * q, br * NS_ + q] = 10.0
            m[br * P + 2 * q + 1, br * NS_ + q] = 1.0
            offs[br * NS_ + q] = 100.0 * q
        m[br * P + 24, br * NS_ + NQ] = 1.0
        offs[br * NS_ + NQ] = 1200.0
    return m, offs.reshape(1, SLOTS)


_PM, _POFF = _pair_const()
_NBLK = 16
_BS = B // _NBLK


def _pair_body(x_ref, m_ref, off_ref, o_ref):
    xf = x_ref[...].astype(jnp.float32)
    pr = jnp.dot(xf, m_ref[...], preferred_element_type=jnp.float32)
    v3 = pr.reshape(_BS // 128, 128, SLOTS)
    eye = jnp.eye(128, dtype=jnp.float32)
    tr = lax.dot_general(v3, eye, (((1,), (0,)), ((), ())),
                         preferred_element_type=jnp.float32)
    o_ref[...] = (tr + off_ref[...].reshape(1, SLOTS, 1)).astype(jnp.int32)


def _pair(x2):
    return pl.pallas_call(
        _pair_body,
        grid=(_NBLK,),
        in_specs=[
            pl.BlockSpec((_BS, 2 * P), lambda i: (i, 0)),
            pl.BlockSpec((2 * P, SLOTS), lambda i: (0, 0)),
            pl.BlockSpec((1, SLOTS), lambda i: (0, 0)),
        ],
        out_specs=pl.BlockSpec((_BS // 128, SLOTS, 128), lambda i: (i, 0, 0)),
        out_shape=jax.ShapeDtypeStruct((B // 128, SLOTS, 128), jnp.int32),
    )(x2, jnp.asarray(_PM), jnp.asarray(_POFF))


def _pair_expand_const() -> np.ndarray:
    j = np.arange(RT)
    q, a, b = j // 100, (j % 100) // 10, j % 10
    col1 = np.where(j < 1200, 20 * q + a, np.where(j < 1210, 240 + (j - 1200), -1))
    col2 = np.where(j < 1200, 20 * q + 10 + b, -1)
    c = np.arange(256)[:, None]
    return ((c == col1[None, :]).astype(np.float32)
            + (c == col2[None, :]).astype(np.float32))


_AT = _pair_expand_const()


def _prep_body(emb_ref, loc_ref, lng_ref, lnb_ref, w1_ref, at_ref, p_ref):
    ct = lax.broadcasted_iota(jnp.int32, (16, 256), 0)
    rt = lax.broadcasted_iota(jnp.int32, (16, 256), 1)
    oh_t = (ct == rt % 10).astype(jnp.float32)
    cp = lax.broadcasted_iota(jnp.int32, (32, 256), 0)
    rp = lax.broadcasted_iota(jnp.int32, (32, 256), 1)
    oh_p = (cp == rp // 10).astype(jnp.float32)
    e = (jnp.dot(emb_ref[...], oh_t, preferred_element_type=jnp.float32)
         + jnp.dot(loc_ref[...], oh_p, preferred_element_type=jnp.float32))
    mu = jnp.mean(e, axis=0, keepdims=True)
    var = jnp.mean((e - mu) ** 2, axis=0, keepdims=True)
    ln = (e - mu) * lax.rsqrt(var + EPS) * lng_ref[...] + lnb_ref[...]
    v = lax.dot_general(w1_ref[...], ln, (((0,), (0,)), ((), ()))) * (1.0 / P)
    p_ref[...] = jnp.dot(v, at_ref[...], preferred_element_type=jnp.float32)


def _prep(emb, loc_emb, ln_g, ln_b, W1):
    emb_t = jnp.pad(emb.T, ((0, 0), (0, 6)))
    loc_t = jnp.pad(loc_emb.T, ((0, 0), (0, 7)))
    return pl.pallas_call(
        _prep_body,
        out_shape=jax.ShapeDtypeStruct((D, RT), jnp.float32),
    )(emb_t, loc_t, ln_g.reshape(D, 1), ln_b.reshape(D, 1), W1,
      jnp.asarray(_AT))


def _sc_body(xp_hbm, u_hbm, h1_hbm, h2_hbm, xp_v, u_v, h1_v, h2_v):
    wid = lax.axis_index("s") * NC + lax.axis_index("c")
    pltpu.sync_copy(xp_hbm.at[pl.ds(wid * (BPW * SLOTS), BPW * SLOTS)], xp_v)
    pltpu.sync_copy(u_hbm, u_v)

    @plsc.parallel_loop(0, G, 1, unroll=2)
    def gbody(g):
        m_off = (g // 8) * (SLOTS * 128) + (g % 8) * L
        for br in range(2):
            accs = [jnp.zeros((16,), jnp.float32) for _ in range(D)]
            for j in range(NS_):
                slot = br * NS_ + j
                xv = xp_v[pl.ds(m_off + slot * 128, L)]
                for dd in range(D):
                    accs[dd] = accs[dd] + plsc.load_gather(u_v, [xv + dd * RT])
            hv = h1_v if br == 0 else h2_v
            for dd in range(D):
                hv[pl.ds(dd * BPW + g * L, L)] = accs[dd]
    for dd in range(D):
        pltpu.sync_copy(h1_v.at[pl.ds(dd * BPW, BPW)], h1_hbm.at[wid * D + dd])
        pltpu.sync_copy(h2_v.at[pl.ds(dd * BPW, BPW)], h2_hbm.at[wid * D + dd])


_sc_gather = functools.partial(
    pl.kernel,
    out_type=(jax.ShapeDtypeStruct((NW * D, BPW), jnp.float32),
              jax.ShapeDtypeStruct((NW * D, BPW), jnp.float32)),
    mesh=plsc.VectorSubcoreMesh(core_axis_name="c", subcore_axis_name="s"),
    compiler_params=pltpu.CompilerParams(needs_layout_passes=False),
    scratch_types=[
        pltpu.VMEM((BPW * SLOTS,), jnp.int32),
        pltpu.VMEM((D * RT,), jnp.float32),
        pltpu.VMEM((D * BPW,), jnp.float32),
        pltpu.VMEM((D * BPW,), jnp.float32),
    ],
)(_sc_body)


_EC = (np.arange(NW * D)[None, :] % D == np.arange(D)[:, None]).astype(np.float32)
_EG = (np.arange(NW * D)[None, :] // D == np.arange(NW)[:, None]).astype(np.float32)


def _fin_body(h1_ref, h2_ref, bng_ref, wh_ref, bh_ref, ec_ref, eg_ref, o_ref):
    ec = ec_ref[...]
    h1 = h1_ref[...]
    h2 = h2_ref[...]

    def coeffs(h):
        s = jnp.dot(ec, h, preferred_element_type=jnp.float32)
        ss = jnp.dot(ec, h * h, preferred_element_type=jnp.float32)
        m = jnp.sum(s, axis=1, keepdims=True) * (1.0 / B)
        v = jnp.sum(ss, axis=1, keepdims=True) * (1.0 / B) - m * m
        a = lax.rsqrt(v + EPS) * bng_ref[...]
        arow = lax.dot_general(ec, a, (((0,), (0,)), ((), ())))
        mrow = lax.dot_general(ec, m, (((0,), (0,)), ((), ())))
        return arow, mrow

    a1, m1 = coeffs(h1)
    a2, m2 = coeffs(h2)
    whrow = lax.dot_general(ec, wh_ref[...], (((0,), (0,)), ((), ())))
    z = ((h1 - m1) * a1 - (h2 - m2) * a2) * (0.5 * whrow)
    y = jnp.dot(eg_ref[...], z, preferred_element_type=jnp.float32) + bh_ref[...]
    o_ref[...] = jax.nn.sigmoid(y)


def _finish(h1, h2, bn_g, Wh, bh):
    return pl.pallas_call(
        _fin_body,
        out_shape=jax.ShapeDtypeStruct((NW, BPW), jnp.float32),
    )(h1, h2, bn_g.reshape(D, 1), Wh.reshape(D, 1), bh.reshape(1, 1),
      jnp.asarray(_EC), jnp.asarray(_EG))


def kernel(x, emb, loc_emb, ln_g, ln_b, W1, b1, bn_g, bn_b, Wh, bh):
    del b1, bn_b
    xp = _pair(x.reshape(B, 2 * P))
    u = _prep(emb, loc_emb, ln_g, ln_b, W1)
    h1, h2 = _sc_gather(xp.reshape(B * SLOTS), u.reshape(D * RT))
    out = _finish(h1, h2, bn_g, Wh, bh)
    return out.reshape(B, 1)


# speedup vs baseline: 1.4399x; 1.4399x over previous
import functools

import jax
import jax.numpy as jnp
import numpy as np
from jax import lax
from jax.experimental import pallas as pl
from jax.experimental.pallas import tpu as pltpu
from jax.experimental.pallas import tpu_sc as plsc

B = 16384
D = 16
P = 25
EPS = 1e-5
NQ = 12
NS_ = 13
RT = 1280
SLOTS = 32

_info = plsc.get_sparse_core_info()
NC, NCS, L = _info.num_cores, _info.num_subcores, _info.num_lanes
NW = NC * NCS
BPW = B // NW
G = BPW // L


def _pair_const():
    m = np.zeros((50, SLOTS), np.float32)
    offs = np.zeros((SLOTS,), np.float32)
    for br in range(2):
        for q in range(NQ):
            m[br * P + 2 * q, br * NS_ + q] = 10.0
            m[br * P + 2 * q + 1, br * NS_ + q] = 1.0
            offs[br * NS_ + q] = 100.0 * q
        m[br * P + 24, br * NS_ + NQ] = 1.0
        offs[br * NS_ + NQ] = 1200.0
    return m, offs.reshape(1, SLOTS)


_PM, _POFF = _pair_const()
_NBLK = 4
_BS = B // _NBLK


def _pair_body(x_ref, m_ref, off_ref, emb_ref, loc_ref, lng_ref, lnb_ref,
               w1e_ref, w1o_ref, at_ref, o_ref, p_ref):
    pr = jnp.dot(x_ref[...].astype(jnp.bfloat16), m_ref[...],
                 preferred_element_type=jnp.float32)
    tr = pr.T + off_ref[...]
    o_ref[...] = tr.astype(jnp.int32)

    @pl.when(pl.program_id(0) == 0)
    def _build_table():
        _prep_compute(emb_ref, loc_ref, lng_ref, lnb_ref, w1e_ref, w1o_ref,
                      at_ref, p_ref)


def _pair(x2, emb, loc_emb, ln_g, ln_b, W1):
    z = lambda i: (0, 0)
    return pl.pallas_call(
        _pair_body,
        grid=(_NBLK,),
        in_specs=[
            pl.BlockSpec((_BS, 2 * P), lambda i: (i, 0)),
            pl.BlockSpec((2 * P, SLOTS), z),
            pl.BlockSpec((SLOTS, 1), z),
            pl.BlockSpec((10, D), z),
            pl.BlockSpec((P, D), z),
            pl.BlockSpec((D, 1), z),
            pl.BlockSpec((D, 1), z),
            pl.BlockSpec((D, D // 2), z),
            pl.BlockSpec((D, D // 2), z),
            pl.BlockSpec((256, RT), z),
        ],
        out_specs=[pl.BlockSpec((SLOTS, _BS), lambda i: (0, i)),
                   pl.BlockSpec((D // 2, RT), z)],
        out_shape=(jax.ShapeDtypeStruct((SLOTS, B), jnp.int32),
                   jax.ShapeDtypeStruct((D // 2, RT), jnp.int32)),
    )(x2, jnp.asarray(_PM, jnp.bfloat16), jnp.asarray(_POFF.reshape(SLOTS, 1)),
      emb, loc_emb, ln_g.reshape(D, 1), ln_b.reshape(D, 1),
      W1[:, 0::2], W1[:, 1::2], jnp.asarray(_AT))


def _pair_expand_const() -> np.ndarray:
    j = np.arange(RT)
    q, a, b = j // 100, (j % 100) // 10, j % 10
    col1 = np.where(j < 1200, 20 * q + a, np.where(j < 1210, 240 + (j - 1200), -1))
    col2 = np.where(j < 1200, 20 * q + 10 + b, -1)
    c = np.arange(256)[:, None]
    return ((c == col1[None, :]).astype(np.float32)
            + (c == col2[None, :]).astype(np.float32))


_AT = _pair_expand_const()


def _prep_compute(emb_ref, loc_ref, lng_ref, lnb_ref, w1e_ref, w1o_ref, at_ref,
                  p_ref):
    ct = lax.broadcasted_iota(jnp.int32, (10, 256), 0)
    rt = lax.broadcasted_iota(jnp.int32, (10, 256), 1)
    oh_t = (ct == rt % 10).astype(jnp.float32)
    cp = lax.broadcasted_iota(jnp.int32, (25, 256), 0)
    rp = lax.broadcasted_iota(jnp.int32, (25, 256), 1)
    oh_p = (cp == rp // 10).astype(jnp.float32)
    e = (lax.dot_general(emb_ref[...], oh_t, (((0,), (0,)), ((), ())),
                         preferred_element_type=jnp.float32)
         + lax.dot_general(loc_ref[...], oh_p, (((0,), (0,)), ((), ())),
                           preferred_element_type=jnp.float32))
    mu = jnp.mean(e, axis=0, keepdims=True)
    var = jnp.mean((e - mu) ** 2, axis=0, keepdims=True)
    ln = (e - mu) * lax.rsqrt(var + EPS) * lng_ref[...] + lnb_ref[...]
    ve = lax.dot_general(w1e_ref[...], ln, (((0,), (0,)), ((), ()))) * (1.0 / P)
    vo = lax.dot_general(w1o_ref[...], ln, (((0,), (0,)), ((), ()))) * (1.0 / P)
    pe = jnp.dot(ve, at_ref[...], preferred_element_type=jnp.float32)
    po = jnp.dot(vo, at_ref[...], preferred_element_type=jnp.float32)
    lo = lax.bitcast_convert_type(pe.astype(jnp.bfloat16),
                                  jnp.uint16).astype(jnp.uint32)
    hi = lax.bitcast_convert_type(po.astype(jnp.bfloat16),
                                  jnp.uint16).astype(jnp.uint32)
    p_ref[...] = lax.bitcast_convert_type(lo | (hi << 16), jnp.int32)


def _sc_body(xp_hbm, u_hbm, h1_hbm, h2_hbm, xp_v, u_v, h1_v, h2_v, sem):
    wid = lax.axis_index("s") * NC + lax.axis_index("c")
    handles = [pltpu.async_copy(xp_hbm.at[s, pl.ds(wid * BPW, BPW)],
                                xp_v.at[pl.ds(s * BPW, BPW)], sem)
               for s in range(2 * NS_)]
    handles += [pltpu.async_copy(u_hbm.at[k], u_v.at[pl.ds(k * RT, RT)], sem)
                for k in range(D // 2)]
    for h in handles:
        h.wait()

    himask = jnp.int32(-65536)

    def gbody(g, carry):
        for br in range(2):
            accs = [jnp.zeros((16,), jnp.float32) for _ in range(D)]
            for j in range(NS_):
                slot = br * NS_ + j
                xv = xp_v[pl.ds(slot * BPW + g * L, L)]
                for k in range(D // 2):
                    w = plsc.load_gather(u_v, [xv + k * RT])
                    ev = plsc.bitcast(lax.shift_left(w, 16), jnp.float32)
                    ov = plsc.bitcast(w & himask, jnp.float32)
                    accs[2 * k] = accs[2 * k] + ev
                    accs[2 * k + 1] = accs[2 * k + 1] + ov
            hv = h1_v if br == 0 else h2_v
            for dd in range(D):
                hv[pl.ds(dd * BPW + g * L, L)] = accs[dd]
        return carry

    lax.fori_loop(0, G, gbody, 0)
    out_handles = []
    for dd in range(D):
        out_handles.append(pltpu.async_copy(
            h1_v.at[pl.ds(dd * BPW, BPW)], h1_hbm.at[wid * D + dd], sem))
        out_handles.append(pltpu.async_copy(
            h2_v.at[pl.ds(dd * BPW, BPW)], h2_hbm.at[wid * D + dd], sem))
    for h in out_handles:
        h.wait()


_sc_gather = functools.partial(
    pl.kernel,
    out_type=(jax.ShapeDtypeStruct((NW * D, BPW), jnp.float32),
              jax.ShapeDtypeStruct((NW * D, BPW), jnp.float32)),
    mesh=plsc.VectorSubcoreMesh(core_axis_name="c", subcore_axis_name="s"),
    compiler_params=pltpu.CompilerParams(needs_layout_passes=False),
    scratch_types=[
        pltpu.VMEM((BPW * SLOTS,), jnp.int32),
        pltpu.VMEM(((D // 2) * RT,), jnp.int32),
        pltpu.VMEM((D * BPW,), jnp.float32),
        pltpu.VMEM((D * BPW,), jnp.float32),
        pltpu.SemaphoreType.DMA,
    ],
)(_sc_body)


_EC = (np.arange(NW * D)[None, :] % D == np.arange(D)[:, None]).astype(np.float32)
_EG = (np.arange(NW * D)[None, :] // D == np.arange(NW)[:, None]).astype(np.float32)


def _fin_body(h1_ref, h2_ref, bng_ref, wh_ref, bh_ref, ec_ref, eg_ref, o_ref):
    ec = ec_ref[...]
    h1 = h1_ref[...]
    h2 = h2_ref[...]

    def coeffs(h):
        s = jnp.dot(ec, h, preferred_element_type=jnp.float32)
        ss = jnp.dot(ec, h * h, preferred_element_type=jnp.float32)
        m = jnp.sum(s, axis=1, keepdims=True) * (1.0 / B)
        v = jnp.sum(ss, axis=1, keepdims=True) * (1.0 / B) - m * m
        a = lax.rsqrt(v + EPS) * bng_ref[...]
        arow = lax.dot_general(ec, a, (((0,), (0,)), ((), ())))
        mrow = lax.dot_general(ec, m, (((0,), (0,)), ((), ())))
        return arow, mrow

    a1, m1 = coeffs(h1)
    a2, m2 = coeffs(h2)
    whrow = lax.dot_general(ec, wh_ref[...], (((0,), (0,)), ((), ())))
    z = ((h1 - m1) * a1 - (h2 - m2) * a2) * (0.5 * whrow)
    y = jnp.dot(eg_ref[...], z, preferred_element_type=jnp.float32) + bh_ref[...]
    o_ref[...] = jax.nn.sigmoid(y)


def _finish(h1, h2, bn_g, Wh, bh):
    return pl.pallas_call(
        _fin_body,
        out_shape=jax.ShapeDtypeStruct((NW, BPW), jnp.float32),
    )(h1, h2, bn_g.reshape(D, 1), Wh.reshape(D, 1), bh.reshape(1, 1),
      jnp.asarray(_EC), jnp.asarray(_EG))


def kernel(x, emb, loc_emb, ln_g, ln_b, W1, b1, bn_g, bn_b, Wh, bh):
    del b1, bn_b
    xp, u = _pair(x.reshape(B, 2 * P), emb, loc_emb, ln_g, ln_b, W1)
    h1, h2 = _sc_gather(xp, u)
    out = _finish(h1, h2, bn_g, Wh, bh)
    return out.reshape(B, 1)


# speedup vs baseline: 1.5704x; 1.0906x over previous
import functools

import jax
import jax.numpy as jnp
import numpy as np
from jax import lax
from jax.experimental import pallas as pl
from jax.experimental.pallas import tpu as pltpu
from jax.experimental.pallas import tpu_sc as plsc

B = 16384
D = 16
P = 25
EPS = 1e-5
NQ = 12
NS_ = 13
RT = 1280
SLOTS = 32

_info = plsc.get_sparse_core_info()
NC, NCS, L = _info.num_cores, _info.num_subcores, _info.num_lanes
NW = NC * NCS
BPW = B // NW
G = BPW // L


def _pair_const():
    m = np.zeros((50, SLOTS), np.float32)
    offs = np.zeros((SLOTS,), np.float32)
    for br in range(2):
        for q in range(NQ):
            m[br * P + 2 * q, br * NS_ + q] = 10.0
            m[br * P + 2 * q + 1, br * NS_ + q] = 1.0
            offs[br * NS_ + q] = 100.0 * q
        m[br * P + 24, br * NS_ + NQ] = 1.0
        offs[br * NS_ + NQ] = 1200.0
    return m, offs.reshape(1, SLOTS)


_PM, _POFF = _pair_const()
_NBLK = 4
_BS = B // _NBLK


def _pair_body(x_ref, m_ref, off_ref, emb_ref, loc_ref, lng_ref, lnb_ref,
               w1e_ref, w1o_ref, at_ref, o_ref, p_ref):
    pr = jnp.dot(x_ref[...], m_ref[...], preferred_element_type=jnp.float32)
    tr = pr.T + off_ref[...]
    o_ref[...] = tr.astype(jnp.int32)

    @pl.when(pl.program_id(0) == 0)
    def _build_table():
        _prep_compute(emb_ref, loc_ref, lng_ref, lnb_ref, w1e_ref, w1o_ref,
                      at_ref, p_ref)


def _pair(x2bf, emb, loc_emb, ln_g, ln_b, W1):
    z = lambda i: (0, 0)
    return pl.pallas_call(
        _pair_body,
        grid=(_NBLK,),
        in_specs=[
            pl.BlockSpec((_BS, 2 * P), lambda i: (i, 0)),
            pl.BlockSpec((2 * P, SLOTS), z),
            pl.BlockSpec((SLOTS, 1), z),
            pl.BlockSpec((10, D), z),
            pl.BlockSpec((P, D), z),
            pl.BlockSpec((D, 1), z),
            pl.BlockSpec((D, 1), z),
            pl.BlockSpec((D, D // 2), z),
            pl.BlockSpec((D, D // 2), z),
            pl.BlockSpec((256, RT), z),
        ],
        out_specs=[pl.BlockSpec((SLOTS, _BS), lambda i: (0, i)),
                   pl.BlockSpec((D // 2, RT), z)],
        out_shape=(jax.ShapeDtypeStruct((SLOTS, B), jnp.int32),
                   jax.ShapeDtypeStruct((D // 2, RT), jnp.int32)),
    )(x2bf, jnp.asarray(_PM, jnp.bfloat16), jnp.asarray(_POFF.reshape(SLOTS, 1)),
      emb, loc_emb, ln_g.reshape(D, 1), ln_b.reshape(D, 1),
      W1[:, 0::2], W1[:, 1::2], jnp.asarray(_AT))


def _pair_expand_const() -> np.ndarray:
    j = np.arange(RT)
    q, a, b = j // 100, (j % 100) // 10, j % 10
    col1 = np.where(j < 1200, 20 * q + a, np.where(j < 1210, 240 + (j - 1200), -1))
    col2 = np.where(j < 1200, 20 * q + 10 + b, -1)
    c = np.arange(256)[:, None]
    return ((c == col1[None, :]).astype(np.float32)
            + (c == col2[None, :]).astype(np.float32))


_AT = _pair_expand_const()


def _prep_compute(emb_ref, loc_ref, lng_ref, lnb_ref, w1e_ref, w1o_ref, at_ref,
                  p_ref):
    ct = lax.broadcasted_iota(jnp.int32, (10, 256), 0)
    rt = lax.broadcasted_iota(jnp.int32, (10, 256), 1)
    oh_t = (ct == rt % 10).astype(jnp.float32)
    cp = lax.broadcasted_iota(jnp.int32, (25, 256), 0)
    rp = lax.broadcasted_iota(jnp.int32, (25, 256), 1)
    oh_p = (cp == rp // 10).astype(jnp.float32)
    e = (lax.dot_general(emb_ref[...], oh_t, (((0,), (0,)), ((), ())),
                         preferred_element_type=jnp.float32)
         + lax.dot_general(loc_ref[...], oh_p, (((0,), (0,)), ((), ())),
                           preferred_element_type=jnp.float32))
    mu = jnp.mean(e, axis=0, keepdims=True)
    var = jnp.mean((e - mu) ** 2, axis=0, keepdims=True)
    ln = (e - mu) * lax.rsqrt(var + EPS) * lng_ref[...] + lnb_ref[...]
    ve = lax.dot_general(w1e_ref[...], ln, (((0,), (0,)), ((), ()))) * (1.0 / P)
    vo = lax.dot_general(w1o_ref[...], ln, (((0,), (0,)), ((), ()))) * (1.0 / P)
    pe = jnp.dot(ve, at_ref[...], preferred_element_type=jnp.float32)
    po = jnp.dot(vo, at_ref[...], preferred_element_type=jnp.float32)
    lo = lax.bitcast_convert_type(pe.astype(jnp.bfloat16),
                                  jnp.uint16).astype(jnp.uint32)
    hi = lax.bitcast_convert_type(po.astype(jnp.bfloat16),
                                  jnp.uint16).astype(jnp.uint32)
    p_ref[...] = lax.bitcast_convert_type(lo | (hi << 16), jnp.int32)


def _sc_body(xp_hbm, u_hbm, h1_hbm, h2_hbm, xp_v, u_v, h1_v, h2_v, sem):
    wid = lax.axis_index("s") * NC + lax.axis_index("c")
    handles = [pltpu.async_copy(xp_hbm.at[s, pl.ds(wid * BPW, BPW)],
                                xp_v.at[pl.ds(s * BPW, BPW)], sem)
               for s in range(2 * NS_)]
    handles += [pltpu.async_copy(u_hbm.at[k], u_v.at[pl.ds(k * RT, RT)], sem)
                for k in range(D // 2)]
    for h in handles:
        h.wait()

    himask = jnp.int32(-65536)

    def gbody(g, carry):
        for br in range(2):
            accs = [jnp.zeros((16,), jnp.float32) for _ in range(D)]
            for j in range(NS_):
                slot = br * NS_ + j
                xv = xp_v[pl.ds(slot * BPW + g * L, L)]
                for k in range(D // 2):
                    w = plsc.load_gather(u_v, [xv + k * RT])
                    ev = plsc.bitcast(lax.shift_left(w, 16), jnp.float32)
                    ov = plsc.bitcast(w & himask, jnp.float32)
                    accs[2 * k] = accs[2 * k] + ev
                    accs[2 * k + 1] = accs[2 * k + 1] + ov
            hv = h1_v if br == 0 else h2_v
            for dd in range(D):
                hv[pl.ds(dd * BPW + g * L, L)] = accs[dd]
        return carry

    lax.fori_loop(0, G, gbody, 0)
    out_handles = []
    for dd in range(D):
        out_handles.append(pltpu.async_copy(
            h1_v.at[pl.ds(dd * BPW, BPW)], h1_hbm.at[wid * D + dd], sem))
        out_handles.append(pltpu.async_copy(
            h2_v.at[pl.ds(dd * BPW, BPW)], h2_hbm.at[wid * D + dd], sem))
    for h in out_handles:
        h.wait()


_sc_gather = functools.partial(
    pl.kernel,
    out_type=(jax.ShapeDtypeStruct((NW * D, BPW), jnp.float32),
              jax.ShapeDtypeStruct((NW * D, BPW), jnp.float32)),
    mesh=plsc.VectorSubcoreMesh(core_axis_name="c", subcore_axis_name="s"),
    compiler_params=pltpu.CompilerParams(needs_layout_passes=False),
    scratch_types=[
        pltpu.VMEM((BPW * SLOTS,), jnp.int32),
        pltpu.VMEM(((D // 2) * RT,), jnp.int32),
        pltpu.VMEM((D * BPW,), jnp.float32),
        pltpu.VMEM((D * BPW,), jnp.float32),
        pltpu.SemaphoreType.DMA,
    ],
)(_sc_body)


_EC = (np.arange(NW * D)[None, :] % D == np.arange(D)[:, None]).astype(np.float32)
_EG = (np.arange(NW * D)[None, :] // D == np.arange(NW)[:, None]).astype(np.float32)


def _fin_body(h1_ref, h2_ref, bng_ref, wh_ref, bh_ref, ec_ref, eg_ref, o_ref):
    ec = ec_ref[...]
    h1 = h1_ref[...]
    h2 = h2_ref[...]

    def coeffs(h):
        s = jnp.dot(ec, h, preferred_element_type=jnp.float32)
        ss = jnp.dot(ec, h * h, preferred_element_type=jnp.float32)
        m = jnp.sum(s, axis=1, keepdims=True) * (1.0 / B)
        v = jnp.sum(ss, axis=1, keepdims=True) * (1.0 / B) - m * m
        a = lax.rsqrt(v + EPS) * bng_ref[...]
        arow = lax.dot_general(ec, a, (((0,), (0,)), ((), ())))
        mrow = lax.dot_general(ec, m, (((0,), (0,)), ((), ())))
        return arow, mrow

    a1, m1 = coeffs(h1)
    a2, m2 = coeffs(h2)
    whrow = lax.dot_general(ec, wh_ref[...], (((0,), (0,)), ((), ())))
    z = ((h1 - m1) * a1 - (h2 - m2) * a2) * (0.5 * whrow)
    y = jnp.dot(eg_ref[...], z, preferred_element_type=jnp.float32) + bh_ref[...]
    o_ref[...] = jax.nn.sigmoid(y)


def _finish(h1, h2, bn_g, Wh, bh):
    return pl.pallas_call(
        _fin_body,
        out_shape=jax.ShapeDtypeStruct((NW, BPW), jnp.float32),
    )(h1, h2, bn_g.reshape(D, 1), Wh.reshape(D, 1), bh.reshape(1, 1),
      jnp.asarray(_EC), jnp.asarray(_EG))


def kernel(x, emb, loc_emb, ln_g, ln_b, W1, b1, bn_g, bn_b, Wh, bh):
    del b1, bn_b
    xp, u = _pair(x.reshape(B, 2 * P).astype(jnp.bfloat16),
                  emb, loc_emb, ln_g, ln_b, W1)
    h1, h2 = _sc_gather(xp, u)
    out = _finish(h1, h2, bn_g, Wh, bh)
    return out.reshape(B, 1)
